# baseline (speedup 1.0000x reference)
import functools

import jax
import jax.numpy as jnp
from jax import lax
from jax.experimental import pallas as pl
from jax.experimental.pallas import tpu as pltpu
from jax.experimental.pallas import tpu_sc as plsc

NUM_Z = 4
Z_VOCAB = 100000
NUM_B = 4
B_VOCAB = 1000
ED = 64
IMG = 512
BATCH = 16384
OUT_D = (NUM_Z + NUM_B) * ED + IMG

NC = 2
NS = 16
NW = NC * NS
BPW = BATCH // NW
CH = 128
NCH = BPW // CH
NARM = NUM_Z + NUM_B
RB = 8
NRB = BATCH // RB
RBW = BPW // RB
RBC = CH // RB

NBUF = 4
DEPTH = 3
NTASK = NARM * NCH
PHI_CHUNKS = 8
PHI_HR = 4096 // PHI_CHUNKS
NWIDX = NTASK + 4 * PHI_CHUNKS


def _body(idx_hbm, widx_hbm, phi_hbm, zf_hbm, bf_hbm, out_hbm,
          idx_v, widx_v, rows_v, phi_v, gsem, wsem, psem, ssem):
    wid = lax.axis_index("s") * NC + lax.axis_index("c")

    pltpu.sync_copy(idx_hbm.at[wid], idx_v)
    pltpu.sync_copy(widx_hbm.at[wid], widx_v)

    def gather(t, s):
        a, c = divmod(t, NCH)
        table = zf_hbm if a < NUM_Z else bf_hbm
        return pltpu.async_copy(table.at[idx_v.at[c, a]], rows_v.at[s], gsem.at[s])

    def write(t, s):
        return pltpu.async_copy(rows_v.at[s], out_hbm.at[widx_v.at[t]], wsem.at[s])

    gd = [None] * NBUF
    wd = [None] * NBUF
    for t in range(DEPTH):
        gd[t % NBUF] = gather(t, t % NBUF)
    for t in range(NTASK):
        s = t % NBUF
        gd[s].wait()
        wd[s] = write(t, s)
        tn = t + DEPTH
        if tn < NTASK:
            sn = tn % NBUF
            if wd[sn] is not None:
                wd[sn].wait()
            gd[sn] = gather(tn, sn)
    for s in range(NBUF):
        if wd[s] is not None:
            wd[s].wait()

    pin = [None, None]
    pout = [None] * (2 * 4)
    for c in range(PHI_CHUNKS):
        s = c % 2
        for h in range(4):
            if pout[4 * s + h] is not None:
                pout[4 * s + h].wait()
        pin[s] = pltpu.async_copy(
            phi_hbm.at[pl.ds(wid * 4096 + c * PHI_HR, PHI_HR)], phi_v.at[s], psem.at[s]
        )
        pin[s].wait()
        for h in range(4):
            pout[4 * s + h] = pltpu.async_copy(
                phi_v.at[s, pl.ds(h * CH, CH)],
                out_hbm.at[widx_v.at[NTASK + 4 * c + h]],
                ssem.at[s],
            )
    for d in pout:
        if d is not None:
            d.wait()


@jax.jit
def _run(idx, widx, phi2, z_flat, b_flat):
    mesh = plsc.VectorSubcoreMesh(
        core_axis_name="c", subcore_axis_name="s", num_cores=NC, num_subcores=NS
    )
    return pl.kernel(
        _body,
        out_type=jax.ShapeDtypeStruct((BATCH * OUT_D // ED, ED), jnp.float32),
        mesh=mesh,
        scratch_types=[
            pltpu.VMEM((NCH, NARM, CH), jnp.int32),
            pltpu.VMEM((NWIDX, CH), jnp.int32),
            pltpu.VMEM((NBUF, CH, ED), jnp.float32),
            pltpu.VMEM((2, PHI_HR, ED), jnp.float32),
            pltpu.SemaphoreType.DMA((NBUF,)),
            pltpu.SemaphoreType.DMA((NBUF,)),
            pltpu.SemaphoreType.DMA((2,)),
            pltpu.SemaphoreType.DMA((2,)),
        ],
        compiler_params=pltpu.CompilerParams(use_tc_tiling_on_sc=False),
    )(idx, widx, phi2, z_flat, b_flat)


def _scatter_indices():
    w = jnp.arange(NW, dtype=jnp.int32)[:, None, None]
    i = jnp.arange(CH, dtype=jnp.int32)[None, None, :]
    t = jnp.arange(NTASK, dtype=jnp.int32)[None, :, None]
    a, c = t // NCH, t % NCH
    emb = (w * RBW + c * RBC + i // RB) * 128 + (a // 2) * 16 + (i % RB) * 2 + a % 2
    j = jnp.arange(4 * PHI_CHUNKS, dtype=jnp.int32)[None, :, None]
    g = j * CH + i
    phi = (w * RBW + g // ED) * 128 + ED + ((g // 16) % 4) * 16 + ((g // 2) % 8) * 2 + g % 2
    return jnp.concatenate([emb, phi], axis=1)


def kernel(z, beta, phi_x, Z_tables, B_tables):
    zoff = jnp.arange(NUM_Z, dtype=jnp.int32) * Z_VOCAB
    boff = jnp.arange(NUM_B, dtype=jnp.int32) * B_VOCAB
    zi = (z.astype(jnp.int32) + zoff[None, :]).T
    bi = (beta.astype(jnp.int32) + boff[None, :]).T
    idx8 = jnp.concatenate([zi, bi], axis=0)
    idx = (
        idx8.reshape(NARM, BATCH // CH, CH)
        .transpose(1, 0, 2)
        .reshape(NW, NCH, NARM, CH)
    )
    widx = _scatter_indices()
    phi2 = (
        phi_x.reshape(NRB, RB, IMG // 128, 2, ED)
        .transpose(0, 2, 1, 3, 4)
        .reshape(NRB * (IMG // 128) * RB * 2, ED)
    )
    out2 = _run(
        idx,
        widx,
        phi2,
        Z_tables.reshape(NUM_Z * Z_VOCAB, ED),
        B_tables.reshape(NUM_B * B_VOCAB, ED),
    )
    return (
        out2.reshape(NRB, OUT_D // 128, RB, 2, ED)
        .transpose(0, 2, 1, 3, 4)
        .reshape(BATCH, OUT_D)
    )


# speedup vs baseline: 2.5206x; 2.5206x over previous
import functools

import jax
import jax.numpy as jnp
from jax import lax
from jax.experimental import pallas as pl
from jax.experimental.pallas import tpu as pltpu
from jax.experimental.pallas import tpu_sc as plsc

NUM_Z = 4
Z_VOCAB = 100000
NUM_B = 4
B_VOCAB = 1000
ED = 64
BATCH = 16384
EMB_D = (NUM_Z + NUM_B) * ED

NC = 2
NS = 16
NW = NC * NS
BPW = BATCH // NW
CH = 128
NCH = BPW // CH
NARM = NUM_Z + NUM_B

NBUF = 4
DEPTH = 3
NTASK = NARM * NCH


def _body(idx_hbm, z0_hbm, z1_hbm, z2_hbm, z3_hbm, bf_hbm, out_hbm,
          idx_v, rows_v, gsem, wsem):
    wid = lax.axis_index("s") * NC + lax.axis_index("c")
    base = wid * BPW
    tables = [z0_hbm, z1_hbm, z2_hbm, z3_hbm]

    pltpu.sync_copy(idx_hbm.at[wid], idx_v)

    def gather(t, s):
        a, c = divmod(t, NCH)
        table = tables[a] if a < NUM_Z else bf_hbm
        return pltpu.async_copy(table.at[idx_v.at[c, a]], rows_v.at[s], gsem.at[s])

    def write(t, s):
        a, c = divmod(t, NCH)
        return pltpu.async_copy(
            rows_v.at[s],
            out_hbm.at[pl.ds(base + c * CH, CH), pl.ds(a * ED, ED)],
            wsem.at[s],
        )

    gd = [None] * NBUF
    wd = [None] * NBUF
    for t in range(DEPTH):
        gd[t % NBUF] = gather(t, t % NBUF)
    for t in range(NTASK):
        s = t % NBUF
        gd[s].wait()
        wd[s] = write(t, s)
        tn = t + DEPTH
        if tn < NTASK:
            sn = tn % NBUF
            if wd[sn] is not None:
                wd[sn].wait()
            gd[sn] = gather(tn, sn)
    for s in range(NBUF):
        if wd[s] is not None:
            wd[s].wait()


@jax.jit
def _run(idx, z0, z1, z2, z3, b_flat):
    mesh = plsc.VectorSubcoreMesh(
        core_axis_name="c", subcore_axis_name="s", num_cores=NC, num_subcores=NS
    )
    return pl.kernel(
        _body,
        out_type=jax.ShapeDtypeStruct((BATCH, EMB_D), jnp.float32),
        mesh=mesh,
        scratch_types=[
            pltpu.VMEM((NCH, NARM, CH), jnp.int32),
            pltpu.VMEM((NBUF, CH, ED), jnp.float32),
            pltpu.SemaphoreType.DMA((NBUF,)),
            pltpu.SemaphoreType.DMA((NBUF,)),
        ],
        compiler_params=pltpu.CompilerParams(use_tc_tiling_on_sc=False),
    )(idx, z0, z1, z2, z3, b_flat)


def kernel(z, beta, phi_x, Z_tables, B_tables):
    boff = jnp.arange(NUM_B, dtype=jnp.int32) * B_VOCAB
    zi = z.astype(jnp.int32).T
    bi = (beta.astype(jnp.int32) + boff[None, :]).T
    idx8 = jnp.concatenate([zi, bi], axis=0)
    idx = (
        idx8.reshape(NARM, BATCH // CH, CH)
        .transpose(1, 0, 2)
        .reshape(NW, NCH, NARM, CH)
    )
    emb = _run(
        idx,
        Z_tables[0],
        Z_tables[1],
        Z_tables[2],
        Z_tables[3],
        B_tables.reshape(NUM_B * B_VOCAB, ED),
    )
    return jnp.concatenate([emb, phi_x], axis=1)


# speedup vs baseline: 3.5461x; 1.4068x over previous
import functools

import jax
import jax.numpy as jnp
from jax import lax
from jax.experimental import pallas as pl
from jax.experimental.pallas import tpu as pltpu
from jax.experimental.pallas import tpu_sc as plsc

NUM_Z = 4
Z_VOCAB = 100000
NUM_B = 4
B_VOCAB = 1000
ED = 64
BATCH = 16384
EMB_D = (NUM_Z + NUM_B) * ED

NC = 2
NS = 16
NW = NC * NS
BPW = BATCH // NW
CH = 128
NCH = BPW // CH
NARM = NUM_Z + NUM_B

NBUF = 4
DEPTH = 3
NTASK = NARM * NCH


def _body(idx_hbm, zf_hbm, bf_hbm, out_hbm, idx_v, rows_v, gsem, wsem):
    wid = lax.axis_index("s") * NC + lax.axis_index("c")
    base = wid * BPW

    pltpu.sync_copy(idx_hbm.at[wid], idx_v)

    def gather(t, s):
        a, c = divmod(t, NCH)
        table = zf_hbm if a < NUM_Z else bf_hbm
        return pltpu.async_copy(table.at[idx_v.at[c, a]], rows_v.at[s], gsem.at[s])

    def write(t, s):
        a, c = divmod(t, NCH)
        return pltpu.async_copy(
            rows_v.at[s],
            out_hbm.at[pl.ds(base + c * CH, CH), pl.ds(a * ED, ED)],
            wsem.at[s],
        )

    gd = [None] * NBUF
    wd = [None] * NBUF
    for t in range(DEPTH):
        gd[t % NBUF] = gather(t, t % NBUF)
    for t in range(NTASK):
        s = t % NBUF
        gd[s].wait()
        wd[s] = write(t, s)
        tn = t + DEPTH
        if tn < NTASK:
            sn = tn % NBUF
            if wd[sn] is not None:
                wd[sn].wait()
            gd[sn] = gather(tn, sn)
    for s in range(NBUF):
        if wd[s] is not None:
            wd[s].wait()


@jax.jit
def _run(idx, z_flat, b_flat):
    mesh = plsc.VectorSubcoreMesh(
        core_axis_name="c", subcore_axis_name="s", num_cores=NC, num_subcores=NS
    )
    return pl.kernel(
        _body,
        out_type=jax.ShapeDtypeStruct((BATCH, EMB_D), jnp.float32),
        mesh=mesh,
        scratch_types=[
            pltpu.VMEM((NCH, NARM, CH), jnp.int32),
            pltpu.VMEM((NBUF, CH, ED), jnp.float32),
            pltpu.SemaphoreType.DMA((NBUF,)),
            pltpu.SemaphoreType.DMA((NBUF,)),
        ],
        compiler_params=pltpu.CompilerParams(use_tc_tiling_on_sc=False),
    )(idx, z_flat, b_flat)


def kernel(z, beta, phi_x, Z_tables, B_tables):
    zoff = jnp.arange(NUM_Z, dtype=jnp.int32) * Z_VOCAB
    boff = jnp.arange(NUM_B, dtype=jnp.int32) * B_VOCAB
    zi = (z.astype(jnp.int32) + zoff[None, :]).T
    bi = (beta.astype(jnp.int32) + boff[None, :]).T
    idx8 = jnp.concatenate([zi, bi], axis=0)
    idx = (
        idx8.reshape(NARM, BATCH // CH, CH)
        .transpose(1, 0, 2)
        .reshape(NW, NCH, NARM, CH)
    )
    emb = _run(
        idx,
        Z_tables.reshape(NUM_Z * Z_VOCAB, ED),
        B_tables.reshape(NUM_B * B_VOCAB, ED),
    )
    return jnp.concatenate([emb, phi_x], axis=1)
